# baseline (speedup 1.0000x reference)
import functools

import jax
import jax.numpy as jnp
from jax import lax
from jax.experimental import pallas as pl
from jax.experimental.pallas import tpu as pltpu
from jax.experimental.pallas import tpu_sc as plsc

F32 = jnp.float32

NC = 2
NS = 16
NW = NC * NS

CH = 400


def _mm_body(x_ref, w_ref, o_ref):
    o_ref[...] = jnp.dot(x_ref[...], w_ref[...], preferred_element_type=F32)


def _stage_a(x, w1a_x, blk=2000):
    n = x.shape[0]
    d = w1a_x.shape[1]
    return pl.pallas_call(
        _mm_body,
        grid=(n // blk,),
        in_specs=[
            pl.BlockSpec((blk, x.shape[1]), lambda i: (i, 0)),
            pl.BlockSpec(w1a_x.shape, lambda i: (0, 0)),
        ],
        out_specs=pl.BlockSpec((blk, d), lambda i: (i, 0)),
        out_shape=jax.ShapeDtypeStruct((n, d), F32),
    )(x, w1a_x)


def _gather_body(n_chunks, per_w, ch, lo, xa_hbm, row_hbm, g_hbm, idx_v,
                 rows_v, sem):
    cid = lax.axis_index("c")
    sid = lax.axis_index("s")
    wid = sid * NC + cid
    base_w = wid * per_w

    def body(i, carry):
        off = base_w + i * ch
        pltpu.sync_copy(row_hbm.at[pl.ds(lo + off, ch)], idx_v)
        pltpu.async_copy(xa_hbm.at[idx_v], rows_v, sem).wait()
        pltpu.sync_copy(rows_v, g_hbm.at[pl.ds(off, ch)])
        return carry

    lax.fori_loop(0, n_chunks, body, 0)


def _stage_b(xa, row, lo, ne):
    d = xa.shape[1]
    per_w = ne // NW
    ch = 1000 if per_w % 1000 == 0 else CH
    assert per_w * NW == ne and per_w % ch == 0
    mesh = plsc.VectorSubcoreMesh(core_axis_name="c", subcore_axis_name="s")
    kern = pl.kernel(
        functools.partial(_gather_body, per_w // ch, per_w, ch, lo),
        out_type=jax.ShapeDtypeStruct((ne, d), F32),
        mesh=mesh,
        scratch_types=[
            pltpu.VMEM((ch,), jnp.int32),
            pltpu.VMEM((ch, d), F32),
            pltpu.SemaphoreType.DMA,
        ],
    )
    return kern(xa, row)


def _edge_body(g_ref, attr_ref, w_ref, b_ref, g1_ref, beta1_ref, o_ref):
    p = g_ref[...] + jnp.dot(attr_ref[...], w_ref[...],
                             preferred_element_type=F32) + b_ref[...]
    m = jnp.mean(p, axis=-1, keepdims=True)
    c = p - m
    v = jnp.mean(c * c, axis=-1, keepdims=True)
    r = c * lax.rsqrt(v + 1e-5) * g1_ref[...] + beta1_ref[...]
    o_ref[...] = jnp.maximum(r, 0.0)


def _stage_c(g, edge_attr, w1a_e, b1a, g1, beta1, lo, blk=2000):
    e, d = g.shape
    k = edge_attr.shape[1]
    lob = lo // blk
    vec = lambda a: a.reshape(1, -1)
    return pl.pallas_call(
        _edge_body,
        grid=(e // blk,),
        in_specs=[
            pl.BlockSpec((blk, d), lambda i: (i, 0)),
            pl.BlockSpec((blk, k), lambda i: (i + lob, 0)),
            pl.BlockSpec((k, d), lambda i: (0, 0)),
            pl.BlockSpec((1, d), lambda i: (0, 0)),
            pl.BlockSpec((1, d), lambda i: (0, 0)),
            pl.BlockSpec((1, d), lambda i: (0, 0)),
        ],
        out_specs=pl.BlockSpec((blk, d), lambda i: (i, 0)),
        out_shape=jax.ShapeDtypeStruct((e, d), F32),
    )(g, edge_attr, w1a_e, vec(b1a), vec(g1), vec(beta1))


def _scatter_body(n_chunks, n_pad, ch, lo, r_hbm, col_hbm, sums_hbm,
                  cnt_hbm, idx0, idx1, r0, r1, ones_v, zb_v, zc_v, acc, cacc,
                  rsem0, rsem1, ssem0, ssem1, csem0, csem1):
    cid = lax.axis_index("c")
    sid = lax.axis_index("s")
    rows_per_tile = n_pad // NS
    tile_lo = sid * rows_per_tile
    zrows = zb_v.shape[0]

    def fill16(ref, val, width):
        def body(i, c):
            ref[i // (width // 16), pl.ds((i % (width // 16)) * 16, 16)] = (
                jnp.full((16,), val, F32))
            return c
        lax.fori_loop(0, ref.shape[0] * (width // 16), body, 0)

    fill16(ones_v, 1.0, 16)
    fill16(zb_v, 0.0, zb_v.shape[1])
    fill16(zc_v, 0.0, 16)

    def zero_acc(k, c):
        pltpu.sync_copy(zb_v, acc.at[pl.ds(tile_lo + k * zrows, zrows)])
        return c

    lax.fori_loop(0, rows_per_tile // zrows, zero_acc, 0)

    @pl.when(cid == 0)
    def _():
        pltpu.sync_copy(zc_v, cacc.at[pl.ds(tile_lo, rows_per_tile)])

    plsc.subcore_barrier()

    base_t = sid * (n_chunks * ch)
    idx_v = (idx0, idx1)
    r_v = (r0, r1)
    rsem = (rsem0, rsem1)
    ssem = (ssem0, ssem1)
    csem = (csem0, csem1)
    dh = r0.shape[1]

    sd = [None, None]
    cd = [None, None]
    for i in range(n_chunks):
        b = i & 1
        base = base_t + i * ch
        if sd[b] is not None:
            sd[b].wait()
            cd[b].wait()
        pltpu.sync_copy(col_hbm.at[pl.ds(lo + base, ch)], idx_v[b])
        pltpu.async_copy(r_hbm.at[pl.ds(base, ch), pl.ds(cid * dh, dh)],
                         r_v[b], rsem[b]).wait()
        sd[b] = pltpu.async_copy(r_v[b], acc.at[idx_v[b]], ssem[b], add=True)
        cd[b] = pltpu.async_copy(ones_v, cacc.at[idx_v[b]], csem[b], add=True)
    for b in range(2):
        if sd[b] is not None:
            sd[b].wait()
            cd[b].wait()
    plsc.subcore_barrier()

    pltpu.sync_copy(acc.at[pl.ds(tile_lo, rows_per_tile)],
                    sums_hbm.at[pl.ds(tile_lo, rows_per_tile),
                                pl.ds(cid * dh, dh)])

    @pl.when(cid == 0)
    def _():
        pltpu.sync_copy(cacc.at[pl.ds(tile_lo, rows_per_tile)],
                        cnt_hbm.at[pl.ds(tile_lo, rows_per_tile),
                                   pl.ds(0, 16)])


def _stage_d(r, col, n_nodes, lo):
    e, d = r.shape
    dh = d // NC
    per_tile = e // NS
    ch = CH
    n_pad = -(-n_nodes // (NS * 32)) * (NS * 32)
    rows_per_tile = n_pad // NS
    assert per_tile % ch == 0
    mesh = plsc.VectorSubcoreMesh(core_axis_name="c", subcore_axis_name="s")
    kern = pl.kernel(
        functools.partial(_scatter_body, per_tile // ch, n_pad, ch, lo),
        out_type=(
            jax.ShapeDtypeStruct((n_pad, d), F32),
            jax.ShapeDtypeStruct((n_pad, 128), F32),
        ),
        mesh=mesh,
        scratch_types=[
            pltpu.VMEM((ch,), jnp.int32),
            pltpu.VMEM((ch,), jnp.int32),
            pltpu.VMEM((ch, dh), F32),
            pltpu.VMEM((ch, dh), F32),
            pltpu.VMEM((ch, 16), F32),
            pltpu.VMEM((rows_per_tile // 4, dh), F32),
            pltpu.VMEM((rows_per_tile, 16), F32),
            pltpu.VMEM_SHARED((n_pad, dh), F32),
            pltpu.VMEM_SHARED((n_pad, 16), F32),
            pltpu.SemaphoreType.DMA,
            pltpu.SemaphoreType.DMA,
            pltpu.SemaphoreType.DMA,
            pltpu.SemaphoreType.DMA,
            pltpu.SemaphoreType.DMA,
            pltpu.SemaphoreType.DMA,
        ],
        compiler_params=pltpu.CompilerParams(use_tc_tiling_on_sc=False),
    )
    sums, cnts = kern(r, col)
    return sums[:n_nodes], cnts[:n_nodes]


def _node_body(nparts, x_ref, *refs):
    (s_refs, c_refs, rest) = (refs[:nparts], refs[nparts:2 * nparts],
                              refs[2 * nparts:])
    (oh_ref, u_ref, eps_ref,
     w1b_ref, b1b_ref, w2ax_ref, w2ag_ref, w2au_ref, b2a_ref,
     g2a_ref, beta2a_ref, w2b_ref, b2b_ref, g2b_ref, beta2b_ref,
     wm_ref, bm_ref, wv_ref, bv_ref, wx_ref, bx_ref,
     out_ref, mu_ref, var_ref) = rest
    def ln(h, gg, bb):
        m = jnp.mean(h, axis=-1, keepdims=True)
        c = h - m
        v = jnp.mean(c * c, axis=-1, keepdims=True)
        return c * lax.rsqrt(v + 1e-5) * gg + bb

    dot = lambda a, b: jnp.dot(a, b, preferred_element_type=F32)

    cnt = sum(jnp.sum(c[:, :16], axis=-1, keepdims=True)
              for c in c_refs) / 16.0
    cnt1 = jnp.maximum(cnt, 1.0)
    has = jnp.minimum(cnt, 1.0)
    sums = s_refs[0][...]
    for s in s_refs[1:]:
        sums = sums + s[...]
    agg = dot(sums / cnt1, w1b_ref[...]) + b1b_ref[...] * has

    uw = dot(u_ref[...], w2au_ref[...])
    h2 = (dot(x_ref[...], w2ax_ref[...]) + dot(agg, w2ag_ref[...])
          + dot(oh_ref[...], uw) + b2a_ref[...])
    h2 = jnp.maximum(ln(h2, g2a_ref[...], beta2a_ref[...]), 0.0)
    h2 = dot(h2, w2b_ref[...]) + b2b_ref[...]
    h2 = jnp.maximum(ln(h2, g2b_ref[...], beta2b_ref[...]), 0.0)
    z_mu = dot(h2, wm_ref[...]) + bm_ref[...]
    z_var = dot(h2, wv_ref[...]) + bv_ref[...]
    z = z_mu + eps_ref[...] * jnp.exp(0.5 * z_var)
    out_ref[...] = dot(z, wx_ref[...]) + bx_ref[...]
    mu_ref[...] = z_mu
    var_ref[...] = z_var


def _stage_e(x, sums_list, cnts_list, onehot, u, eps, w1b, b1b, w2a_x,
             w2a_agg, w2a_u,
             b2a, g2a, beta2a, w2b, b2b, g2b, beta2b, wm, bm, wv, bv, wx, bx,
             blk=2000):
    n, d = x.shape
    lz = wm.shape[1]
    nparts = len(sums_list)
    vec = lambda a: a.reshape(1, -1)
    row_blk = lambda cols: pl.BlockSpec((blk, cols), lambda i: (i, 0))
    full = lambda a: pl.BlockSpec(a.shape, lambda i: tuple(0 for _ in a.shape))
    outs = pl.pallas_call(
        functools.partial(_node_body, nparts),
        grid=(n // blk,),
        in_specs=[
            row_blk(d),
            *[row_blk(d) for _ in range(nparts)],
            *[row_blk(128) for _ in range(nparts)],
            row_blk(onehot.shape[1]),
            full(u),
            row_blk(lz),
            full(w1b), full(vec(b1b)),
            full(w2a_x), full(w2a_agg), full(w2a_u), full(vec(b2a)),
            full(vec(g2a)), full(vec(beta2a)),
            full(w2b), full(vec(b2b)), full(vec(g2b)), full(vec(beta2b)),
            full(wm), full(vec(bm)), full(wv), full(vec(bv)),
            full(wx), full(vec(bx)),
        ],
        out_specs=(row_blk(d), row_blk(lz), row_blk(lz)),
        out_shape=(
            jax.ShapeDtypeStruct((n, d), F32),
            jax.ShapeDtypeStruct((n, lz), F32),
            jax.ShapeDtypeStruct((n, lz), F32),
        ),
    )(x, *sums_list, *cnts_list, onehot, u, eps,
      w1b, vec(b1b), w2a_x, w2a_agg, w2a_u, vec(b2a), vec(g2a), vec(beta2a),
      w2b, vec(b2b), vec(g2b), vec(beta2b), wm, vec(bm), wv, vec(bv),
      wx, vec(bx))
    return outs


def kernel(x, edge_index, edge_attr, u, batch, goal, W1a, b1a, g1, beta1,
           W1b, b1b, W2a, b2a, g2a, beta2a, W2b, b2b, g2b, beta2b,
           Wm, bm, Wv, bv, Wx, bx):
    n, d = x.shape
    row = edge_index[0]
    col = edge_index[1]

    w1a_x = W1a[:d]
    w1a_e = W1a[d:]
    w2a_x = W2a[:d]
    w2a_agg = W2a[d:d + W1b.shape[1]]
    w2a_u = W2a[d + W1b.shape[1]:]

    onehot = (batch[:, None] == jnp.arange(u.shape[0],
                                           dtype=batch.dtype)).astype(F32)
    eps = jax.random.normal(jax.random.key(42), (n, Wm.shape[1]), dtype=F32)

    xa = _stage_a(x, w1a_x)
    e = row.shape[0]
    nparts = 1
    part = e // nparts
    sums_list, cnts_list = [], []
    for k in range(nparts):
        lo = k * part
        g = _stage_b(xa, row, lo, part)
        r = _stage_c(g, edge_attr, w1a_e, b1a, g1, beta1, lo)
        s, c = _stage_d(r, col, n, lo)
        sums_list.append(s)
        cnts_list.append(c)
    out, z_mu, z_var = _stage_e(
        x, sums_list, cnts_list, onehot, u, eps, W1b, b1b, w2a_x, w2a_agg,
        w2a_u, b2a, g2a, beta2a, W2b, b2b, g2b, beta2b, Wm, bm, Wv, bv,
        Wx, bx)
    return (out, z_mu, z_var)


# speedup vs baseline: 1.1115x; 1.1115x over previous
import functools

import jax
import jax.numpy as jnp
from jax import lax
from jax.experimental import pallas as pl
from jax.experimental.pallas import tpu as pltpu
from jax.experimental.pallas import tpu_sc as plsc

F32 = jnp.float32

NC = 2
NS = 16
NW = NC * NS

CH = 400


def _mm_body(x_ref, w_ref, o_ref):
    o_ref[...] = jnp.dot(x_ref[...], w_ref[...], preferred_element_type=F32)


def _stage_a(x, w1a_x, blk=2000):
    n = x.shape[0]
    d = w1a_x.shape[1]
    return pl.pallas_call(
        _mm_body,
        grid=(n // blk,),
        in_specs=[
            pl.BlockSpec((blk, x.shape[1]), lambda i: (i, 0)),
            pl.BlockSpec(w1a_x.shape, lambda i: (0, 0)),
        ],
        out_specs=pl.BlockSpec((blk, d), lambda i: (i, 0)),
        out_shape=jax.ShapeDtypeStruct((n, d), F32),
    )(x, w1a_x)


def _gather_body(n_chunks, per_w, ch, lo, xa_hbm, row_hbm, g_hbm, idx_v,
                 rows_v, sem):
    cid = lax.axis_index("c")
    sid = lax.axis_index("s")
    wid = sid * NC + cid
    base_w = wid * per_w

    def body(i, carry):
        off = base_w + i * ch
        pltpu.sync_copy(row_hbm.at[pl.ds(lo + off, ch)], idx_v)
        pltpu.async_copy(xa_hbm.at[idx_v], rows_v, sem).wait()
        pltpu.sync_copy(rows_v, g_hbm.at[pl.ds(off, ch)])
        return carry

    lax.fori_loop(0, n_chunks, body, 0)


def _stage_b(xa, row, lo, ne):
    d = xa.shape[1]
    per_w = ne // NW
    ch = 1000 if per_w % 1000 == 0 else CH
    assert per_w * NW == ne and per_w % ch == 0
    mesh = plsc.VectorSubcoreMesh(core_axis_name="c", subcore_axis_name="s")
    kern = pl.kernel(
        functools.partial(_gather_body, per_w // ch, per_w, ch, lo),
        out_type=jax.ShapeDtypeStruct((ne, d), F32),
        mesh=mesh,
        scratch_types=[
            pltpu.VMEM((ch,), jnp.int32),
            pltpu.VMEM((ch, d), F32),
            pltpu.SemaphoreType.DMA,
        ],
    )
    return kern(xa, row)


def _edge_body(g_ref, attr_ref, w_ref, b_ref, g1_ref, beta1_ref, o_ref):
    p = g_ref[...] + jnp.dot(attr_ref[...], w_ref[...],
                             preferred_element_type=F32) + b_ref[...]
    m = jnp.mean(p, axis=-1, keepdims=True)
    c = p - m
    v = jnp.mean(c * c, axis=-1, keepdims=True)
    r = c * lax.rsqrt(v + 1e-5) * g1_ref[...] + beta1_ref[...]
    o_ref[...] = jnp.maximum(r, 0.0)


def _stage_c(g, edge_attr, w1a_e, b1a, g1, beta1, lo, blk=2000):
    e, d = g.shape
    k = edge_attr.shape[1]
    lob = lo // blk
    vec = lambda a: a.reshape(1, -1)
    return pl.pallas_call(
        _edge_body,
        grid=(e // blk,),
        in_specs=[
            pl.BlockSpec((blk, d), lambda i: (i, 0)),
            pl.BlockSpec((blk, k), lambda i: (i + lob, 0)),
            pl.BlockSpec((k, d), lambda i: (0, 0)),
            pl.BlockSpec((1, d), lambda i: (0, 0)),
            pl.BlockSpec((1, d), lambda i: (0, 0)),
            pl.BlockSpec((1, d), lambda i: (0, 0)),
        ],
        out_specs=pl.BlockSpec((blk, d), lambda i: (i, 0)),
        out_shape=jax.ShapeDtypeStruct((e, d), F32),
    )(g, edge_attr, w1a_e, vec(b1a), vec(g1), vec(beta1))


def _scatter_body(n_chunks, n_pad, ch, lo, r_hbm, col_hbm, sums_hbm,
                  cnt_hbm, idx0, idx1, r0, r1, ones_v, zb_v, zc_v, acc, cacc,
                  rsem0, rsem1, ssem0, ssem1, csem0, csem1):
    cid = lax.axis_index("c")
    sid = lax.axis_index("s")
    rows_per_tile = n_pad // NS
    tile_lo = sid * rows_per_tile
    zrows = zb_v.shape[0]

    def fill16(ref, val, width):
        def body(i, c):
            ref[i // (width // 16), pl.ds((i % (width // 16)) * 16, 16)] = (
                jnp.full((16,), val, F32))
            return c
        lax.fori_loop(0, ref.shape[0] * (width // 16), body, 0)

    fill16(ones_v, 1.0, 16)
    fill16(zb_v, 0.0, zb_v.shape[1])
    fill16(zc_v, 0.0, 16)

    def zero_acc(k, c):
        pltpu.sync_copy(zb_v, acc.at[pl.ds(tile_lo + k * zrows, zrows)])
        return c

    lax.fori_loop(0, rows_per_tile // zrows, zero_acc, 0)

    @pl.when(cid == 0)
    def _():
        pltpu.sync_copy(zc_v, cacc.at[pl.ds(tile_lo, rows_per_tile)])

    plsc.subcore_barrier()

    base_t = sid * (n_chunks * ch)
    idx_v = (idx0, idx1)
    r_v = (r0, r1)
    rsem = (rsem0, rsem1)
    ssem = (ssem0, ssem1)
    csem = (csem0, csem1)
    dh = r0.shape[1]

    sd = [None, None]
    cd = [None, None]
    for i in range(n_chunks):
        b = i & 1
        base = base_t + i * ch
        if sd[b] is not None:
            sd[b].wait()

            @pl.when(cid == 0)
            def _(b=b):
                cd[b].wait()

        pltpu.sync_copy(col_hbm.at[pl.ds(lo + base, ch)], idx_v[b])
        pltpu.async_copy(r_hbm.at[pl.ds(base, ch), pl.ds(cid * dh, dh)],
                         r_v[b], rsem[b]).wait()
        sd[b] = pltpu.async_copy(r_v[b], acc.at[idx_v[b]], ssem[b], add=True)

        @pl.when(cid == 0)
        def _(b=b):
            cd[b] = pltpu.async_copy(ones_v, cacc.at[idx_v[b]], csem[b],
                                     add=True)

    for b in range(2):
        if sd[b] is not None:
            sd[b].wait()

            @pl.when(cid == 0)
            def _(b=b):
                cd[b].wait()

    plsc.subcore_barrier()

    pltpu.sync_copy(acc.at[pl.ds(tile_lo, rows_per_tile)],
                    sums_hbm.at[pl.ds(tile_lo, rows_per_tile),
                                pl.ds(cid * dh, dh)])

    @pl.when(cid == 0)
    def _():
        pltpu.sync_copy(cacc.at[pl.ds(tile_lo, rows_per_tile)],
                        cnt_hbm.at[pl.ds(tile_lo, rows_per_tile),
                                   pl.ds(0, 16)])


def _stage_d(r, col, n_nodes, lo):
    e, d = r.shape
    dh = d // NC
    per_tile = e // NS
    ch = CH
    n_pad = -(-n_nodes // (NS * 32)) * (NS * 32)
    rows_per_tile = n_pad // NS
    assert per_tile % ch == 0
    mesh = plsc.VectorSubcoreMesh(core_axis_name="c", subcore_axis_name="s")
    kern = pl.kernel(
        functools.partial(_scatter_body, per_tile // ch, n_pad, ch, lo),
        out_type=(
            jax.ShapeDtypeStruct((n_pad, d), F32),
            jax.ShapeDtypeStruct((n_pad, 128), F32),
        ),
        mesh=mesh,
        scratch_types=[
            pltpu.VMEM((ch,), jnp.int32),
            pltpu.VMEM((ch,), jnp.int32),
            pltpu.VMEM((ch, dh), F32),
            pltpu.VMEM((ch, dh), F32),
            pltpu.VMEM((ch, 16), F32),
            pltpu.VMEM((rows_per_tile // 4, dh), F32),
            pltpu.VMEM((rows_per_tile, 16), F32),
            pltpu.VMEM_SHARED((n_pad, dh), F32),
            pltpu.VMEM_SHARED((n_pad, 16), F32),
            pltpu.SemaphoreType.DMA,
            pltpu.SemaphoreType.DMA,
            pltpu.SemaphoreType.DMA,
            pltpu.SemaphoreType.DMA,
            pltpu.SemaphoreType.DMA,
            pltpu.SemaphoreType.DMA,
        ],
        compiler_params=pltpu.CompilerParams(use_tc_tiling_on_sc=False),
    )
    sums, cnts = kern(r, col)
    return sums[:n_nodes], cnts[:n_nodes]


def _node_body(nparts, x_ref, *refs):
    (s_refs, c_refs, rest) = (refs[:nparts], refs[nparts:2 * nparts],
                              refs[2 * nparts:])
    (oh_ref, u_ref, eps_ref,
     w1b_ref, b1b_ref, w2ax_ref, w2ag_ref, w2au_ref, b2a_ref,
     g2a_ref, beta2a_ref, w2b_ref, b2b_ref, g2b_ref, beta2b_ref,
     wm_ref, bm_ref, wv_ref, bv_ref, wx_ref, bx_ref,
     out_ref, mu_ref, var_ref) = rest
    def ln(h, gg, bb):
        m = jnp.mean(h, axis=-1, keepdims=True)
        c = h - m
        v = jnp.mean(c * c, axis=-1, keepdims=True)
        return c * lax.rsqrt(v + 1e-5) * gg + bb

    dot = lambda a, b: jnp.dot(a, b, preferred_element_type=F32)

    cnt = sum(jnp.sum(c[:, :16], axis=-1, keepdims=True)
              for c in c_refs) / 16.0
    cnt1 = jnp.maximum(cnt, 1.0)
    has = jnp.minimum(cnt, 1.0)
    sums = s_refs[0][...]
    for s in s_refs[1:]:
        sums = sums + s[...]
    agg = dot(sums / cnt1, w1b_ref[...]) + b1b_ref[...] * has

    uw = dot(u_ref[...], w2au_ref[...])
    h2 = (dot(x_ref[...], w2ax_ref[...]) + dot(agg, w2ag_ref[...])
          + dot(oh_ref[...], uw) + b2a_ref[...])
    h2 = jnp.maximum(ln(h2, g2a_ref[...], beta2a_ref[...]), 0.0)
    h2 = dot(h2, w2b_ref[...]) + b2b_ref[...]
    h2 = jnp.maximum(ln(h2, g2b_ref[...], beta2b_ref[...]), 0.0)
    z_mu = dot(h2, wm_ref[...]) + bm_ref[...]
    z_var = dot(h2, wv_ref[...]) + bv_ref[...]
    z = z_mu + eps_ref[...] * jnp.exp(0.5 * z_var)
    out_ref[...] = dot(z, wx_ref[...]) + bx_ref[...]
    mu_ref[...] = z_mu
    var_ref[...] = z_var


def _stage_e(x, sums_list, cnts_list, onehot, u, eps, w1b, b1b, w2a_x,
             w2a_agg, w2a_u,
             b2a, g2a, beta2a, w2b, b2b, g2b, beta2b, wm, bm, wv, bv, wx, bx,
             blk=2000):
    n, d = x.shape
    lz = wm.shape[1]
    nparts = len(sums_list)
    vec = lambda a: a.reshape(1, -1)
    row_blk = lambda cols: pl.BlockSpec((blk, cols), lambda i: (i, 0))
    full = lambda a: pl.BlockSpec(a.shape, lambda i: tuple(0 for _ in a.shape))
    outs = pl.pallas_call(
        functools.partial(_node_body, nparts),
        grid=(n // blk,),
        in_specs=[
            row_blk(d),
            *[row_blk(d) for _ in range(nparts)],
            *[row_blk(128) for _ in range(nparts)],
            row_blk(onehot.shape[1]),
            full(u),
            row_blk(lz),
            full(w1b), full(vec(b1b)),
            full(w2a_x), full(w2a_agg), full(w2a_u), full(vec(b2a)),
            full(vec(g2a)), full(vec(beta2a)),
            full(w2b), full(vec(b2b)), full(vec(g2b)), full(vec(beta2b)),
            full(wm), full(vec(bm)), full(wv), full(vec(bv)),
            full(wx), full(vec(bx)),
        ],
        out_specs=(row_blk(d), row_blk(lz), row_blk(lz)),
        out_shape=(
            jax.ShapeDtypeStruct((n, d), F32),
            jax.ShapeDtypeStruct((n, lz), F32),
            jax.ShapeDtypeStruct((n, lz), F32),
        ),
    )(x, *sums_list, *cnts_list, onehot, u, eps,
      w1b, vec(b1b), w2a_x, w2a_agg, w2a_u, vec(b2a), vec(g2a), vec(beta2a),
      w2b, vec(b2b), vec(g2b), vec(beta2b), wm, vec(bm), wv, vec(bv),
      wx, vec(bx))
    return outs


def kernel(x, edge_index, edge_attr, u, batch, goal, W1a, b1a, g1, beta1,
           W1b, b1b, W2a, b2a, g2a, beta2a, W2b, b2b, g2b, beta2b,
           Wm, bm, Wv, bv, Wx, bx):
    n, d = x.shape
    row = edge_index[0]
    col = edge_index[1]

    w1a_x = W1a[:d]
    w1a_e = W1a[d:]
    w2a_x = W2a[:d]
    w2a_agg = W2a[d:d + W1b.shape[1]]
    w2a_u = W2a[d + W1b.shape[1]:]

    onehot = (batch[:, None] == jnp.arange(u.shape[0],
                                           dtype=batch.dtype)).astype(F32)
    eps = jax.random.normal(jax.random.key(42), (n, Wm.shape[1]), dtype=F32)

    xa = _stage_a(x, w1a_x)
    e = row.shape[0]
    nparts = 2
    part = e // nparts
    sums_list, cnts_list = [], []
    for k in range(nparts):
        lo = k * part
        g = _stage_b(xa, row, lo, part)
        r = _stage_c(g, edge_attr, w1a_e, b1a, g1, beta1, lo)
        s, c = _stage_d(r, col, n, lo)
        sums_list.append(s)
        cnts_list.append(c)
    out, z_mu, z_var = _stage_e(
        x, sums_list, cnts_list, onehot, u, eps, W1b, b1b, w2a_x, w2a_agg,
        w2a_u, b2a, g2a, beta2a, W2b, b2b, g2b, beta2b, Wm, bm, Wv, bv,
        Wx, bx)
    return (out, z_mu, z_var)


# speedup vs baseline: 1.1728x; 1.0551x over previous
import functools

import jax
import jax.numpy as jnp
from jax import lax
from jax.experimental import pallas as pl
from jax.experimental.pallas import tpu as pltpu
from jax.experimental.pallas import tpu_sc as plsc

F32 = jnp.float32

NC = 2
NS = 16
NW = NC * NS

CH = 400


def _mm_body(x_ref, w_ref, o_ref):
    o_ref[...] = jnp.dot(x_ref[...], w_ref[...], preferred_element_type=F32)


def _stage_a(x, w1a_x, blk=2000):
    n = x.shape[0]
    d = w1a_x.shape[1]
    return pl.pallas_call(
        _mm_body,
        grid=(n // blk,),
        in_specs=[
            pl.BlockSpec((blk, x.shape[1]), lambda i: (i, 0)),
            pl.BlockSpec(w1a_x.shape, lambda i: (0, 0)),
        ],
        out_specs=pl.BlockSpec((blk, d), lambda i: (i, 0)),
        out_shape=jax.ShapeDtypeStruct((n, d), F32),
    )(x, w1a_x)


def _gather_body(n_chunks, per_w, ch, lo, xa_hbm, row_hbm, g_hbm, idx_v,
                 rows_v, sem):
    cid = lax.axis_index("c")
    sid = lax.axis_index("s")
    wid = sid * NC + cid
    base_w = wid * per_w

    def body(i, carry):
        off = base_w + i * ch
        pltpu.sync_copy(row_hbm.at[pl.ds(lo + off, ch)], idx_v)
        pltpu.async_copy(xa_hbm.at[idx_v], rows_v, sem).wait()
        pltpu.sync_copy(rows_v, g_hbm.at[pl.ds(off, ch)])
        return carry

    lax.fori_loop(0, n_chunks, body, 0)


def _stage_b(xa, row, lo, ne):
    d = xa.shape[1]
    per_w = ne // NW
    ch = 1000 if per_w % 1000 == 0 else CH
    assert per_w * NW == ne and per_w % ch == 0
    mesh = plsc.VectorSubcoreMesh(core_axis_name="c", subcore_axis_name="s")
    kern = pl.kernel(
        functools.partial(_gather_body, per_w // ch, per_w, ch, lo),
        out_type=jax.ShapeDtypeStruct((ne, d), F32),
        mesh=mesh,
        scratch_types=[
            pltpu.VMEM((ch,), jnp.int32),
            pltpu.VMEM((ch, d), F32),
            pltpu.SemaphoreType.DMA,
        ],
    )
    return kern(xa, row)


def _edge_body(g_ref, attr_ref, w_ref, b_ref, g1_ref, beta1_ref, o_ref):
    p = g_ref[...] + jnp.dot(attr_ref[...], w_ref[...],
                             preferred_element_type=F32) + b_ref[...]
    m = jnp.mean(p, axis=-1, keepdims=True)
    c = p - m
    v = jnp.mean(c * c, axis=-1, keepdims=True)
    r = c * lax.rsqrt(v + 1e-5) * g1_ref[...] + beta1_ref[...]
    o_ref[...] = jnp.maximum(r, 0.0)


def _stage_c(g, edge_attr, w1a_e, b1a, g1, beta1, lo, blk=4000):
    e, d = g.shape
    k = edge_attr.shape[1]
    lob = lo // blk
    vec = lambda a: a.reshape(1, -1)
    return pl.pallas_call(
        _edge_body,
        grid=(e // blk,),
        in_specs=[
            pl.BlockSpec((blk, d), lambda i: (i, 0)),
            pl.BlockSpec((blk, k), lambda i: (i + lob, 0)),
            pl.BlockSpec((k, d), lambda i: (0, 0)),
            pl.BlockSpec((1, d), lambda i: (0, 0)),
            pl.BlockSpec((1, d), lambda i: (0, 0)),
            pl.BlockSpec((1, d), lambda i: (0, 0)),
        ],
        out_specs=pl.BlockSpec((blk, d), lambda i: (i, 0)),
        out_shape=jax.ShapeDtypeStruct((e, d), F32),
    )(g, edge_attr, w1a_e, vec(b1a), vec(g1), vec(beta1))


def _scatter_body(n_chunks, n_pad, ch, lo, r_hbm, col_hbm, sums_hbm,
                  cnt_hbm, idx0, idx1, r0, r1, ones_v, zb_v, zc_v, acc, cacc,
                  rsem0, rsem1, ssem0, ssem1, csem0, csem1):
    cid = lax.axis_index("c")
    sid = lax.axis_index("s")
    rows_per_tile = n_pad // NS
    tile_lo = sid * rows_per_tile
    zrows = zb_v.shape[0]

    def fill16(ref, val, width):
        def body(i, c):
            ref[i // (width // 16), pl.ds((i % (width // 16)) * 16, 16)] = (
                jnp.full((16,), val, F32))
            return c
        lax.fori_loop(0, ref.shape[0] * (width // 16), body, 0)

    fill16(ones_v, 1.0, 16)
    fill16(zb_v, 0.0, zb_v.shape[1])
    fill16(zc_v, 0.0, 16)

    def zero_acc(k, c):
        pltpu.sync_copy(zb_v, acc.at[pl.ds(tile_lo + k * zrows, zrows)])
        return c

    lax.fori_loop(0, rows_per_tile // zrows, zero_acc, 0)

    @pl.when(cid == 0)
    def _():
        pltpu.sync_copy(zc_v, cacc.at[pl.ds(tile_lo, rows_per_tile)])

    plsc.subcore_barrier()

    base_t = sid * (n_chunks * ch)
    idx_v = (idx0, idx1)
    r_v = (r0, r1)
    rsem = (rsem0, rsem1)
    ssem = (ssem0, ssem1)
    csem = (csem0, csem1)
    dh = r0.shape[1]

    sd = [None, None]
    cd = [None, None]
    for i in range(n_chunks):
        b = i & 1
        base = base_t + i * ch
        if sd[b] is not None:
            sd[b].wait()

            @pl.when(cid == 0)
            def _(b=b):
                cd[b].wait()

        pltpu.sync_copy(col_hbm.at[pl.ds(lo + base, ch)], idx_v[b])
        pltpu.async_copy(r_hbm.at[pl.ds(base, ch), pl.ds(cid * dh, dh)],
                         r_v[b], rsem[b]).wait()
        sd[b] = pltpu.async_copy(r_v[b], acc.at[idx_v[b]], ssem[b], add=True)

        @pl.when(cid == 0)
        def _(b=b):
            cd[b] = pltpu.async_copy(ones_v, cacc.at[idx_v[b]], csem[b],
                                     add=True)

    for b in range(2):
        if sd[b] is not None:
            sd[b].wait()

            @pl.when(cid == 0)
            def _(b=b):
                cd[b].wait()

    plsc.subcore_barrier()

    pltpu.sync_copy(acc.at[pl.ds(tile_lo, rows_per_tile)],
                    sums_hbm.at[pl.ds(tile_lo, rows_per_tile),
                                pl.ds(cid * dh, dh)])

    @pl.when(cid == 0)
    def _():
        pltpu.sync_copy(cacc.at[pl.ds(tile_lo, rows_per_tile)],
                        cnt_hbm.at[pl.ds(tile_lo, rows_per_tile),
                                   pl.ds(0, 16)])


def _stage_d(r, col, n_nodes, lo):
    e, d = r.shape
    dh = d // NC
    per_tile = e // NS
    ch = CH
    n_pad = -(-n_nodes // (NS * 32)) * (NS * 32)
    rows_per_tile = n_pad // NS
    assert per_tile % ch == 0
    mesh = plsc.VectorSubcoreMesh(core_axis_name="c", subcore_axis_name="s")
    kern = pl.kernel(
        functools.partial(_scatter_body, per_tile // ch, n_pad, ch, lo),
        out_type=(
            jax.ShapeDtypeStruct((n_pad, d), F32),
            jax.ShapeDtypeStruct((n_pad, 128), F32),
        ),
        mesh=mesh,
        scratch_types=[
            pltpu.VMEM((ch,), jnp.int32),
            pltpu.VMEM((ch,), jnp.int32),
            pltpu.VMEM((ch, dh), F32),
            pltpu.VMEM((ch, dh), F32),
            pltpu.VMEM((ch, 16), F32),
            pltpu.VMEM((rows_per_tile // 4, dh), F32),
            pltpu.VMEM((rows_per_tile, 16), F32),
            pltpu.VMEM_SHARED((n_pad, dh), F32),
            pltpu.VMEM_SHARED((n_pad, 16), F32),
            pltpu.SemaphoreType.DMA,
            pltpu.SemaphoreType.DMA,
            pltpu.SemaphoreType.DMA,
            pltpu.SemaphoreType.DMA,
            pltpu.SemaphoreType.DMA,
            pltpu.SemaphoreType.DMA,
        ],
        compiler_params=pltpu.CompilerParams(use_tc_tiling_on_sc=False),
    )
    sums, cnts = kern(r, col)
    return sums[:n_nodes], cnts[:n_nodes]


def _node_body(nparts, x_ref, *refs):
    (s_refs, c_refs, rest) = (refs[:nparts], refs[nparts:2 * nparts],
                              refs[2 * nparts:])
    (oh_ref, u_ref, eps_ref,
     w1b_ref, b1b_ref, w2ax_ref, w2ag_ref, w2au_ref, b2a_ref,
     g2a_ref, beta2a_ref, w2b_ref, b2b_ref, g2b_ref, beta2b_ref,
     wm_ref, bm_ref, wv_ref, bv_ref, wx_ref, bx_ref,
     out_ref, mu_ref, var_ref) = rest
    def ln(h, gg, bb):
        m = jnp.mean(h, axis=-1, keepdims=True)
        c = h - m
        v = jnp.mean(c * c, axis=-1, keepdims=True)
        return c * lax.rsqrt(v + 1e-5) * gg + bb

    dot = lambda a, b: jnp.dot(a, b, preferred_element_type=F32)

    cnt = sum(jnp.sum(c[:, :16], axis=-1, keepdims=True)
              for c in c_refs) / 16.0
    cnt1 = jnp.maximum(cnt, 1.0)
    has = jnp.minimum(cnt, 1.0)
    sums = s_refs[0][...]
    for s in s_refs[1:]:
        sums = sums + s[...]
    agg = dot(sums / cnt1, w1b_ref[...]) + b1b_ref[...] * has

    uw = dot(u_ref[...], w2au_ref[...])
    h2 = (dot(x_ref[...], w2ax_ref[...]) + dot(agg, w2ag_ref[...])
          + dot(oh_ref[...], uw) + b2a_ref[...])
    h2 = jnp.maximum(ln(h2, g2a_ref[...], beta2a_ref[...]), 0.0)
    h2 = dot(h2, w2b_ref[...]) + b2b_ref[...]
    h2 = jnp.maximum(ln(h2, g2b_ref[...], beta2b_ref[...]), 0.0)
    z_mu = dot(h2, wm_ref[...]) + bm_ref[...]
    z_var = dot(h2, wv_ref[...]) + bv_ref[...]
    z = z_mu + eps_ref[...] * jnp.exp(0.5 * z_var)
    out_ref[...] = dot(z, wx_ref[...]) + bx_ref[...]
    mu_ref[...] = z_mu
    var_ref[...] = z_var


def _stage_e(x, sums_list, cnts_list, onehot, u, eps, w1b, b1b, w2a_x,
             w2a_agg, w2a_u,
             b2a, g2a, beta2a, w2b, b2b, g2b, beta2b, wm, bm, wv, bv, wx, bx,
             blk=2000):
    n, d = x.shape
    lz = wm.shape[1]
    nparts = len(sums_list)
    vec = lambda a: a.reshape(1, -1)
    row_blk = lambda cols: pl.BlockSpec((blk, cols), lambda i: (i, 0))
    full = lambda a: pl.BlockSpec(a.shape, lambda i: tuple(0 for _ in a.shape))
    outs = pl.pallas_call(
        functools.partial(_node_body, nparts),
        grid=(n // blk,),
        in_specs=[
            row_blk(d),
            *[row_blk(d) for _ in range(nparts)],
            *[row_blk(128) for _ in range(nparts)],
            row_blk(onehot.shape[1]),
            full(u),
            row_blk(lz),
            full(w1b), full(vec(b1b)),
            full(w2a_x), full(w2a_agg), full(w2a_u), full(vec(b2a)),
            full(vec(g2a)), full(vec(beta2a)),
            full(w2b), full(vec(b2b)), full(vec(g2b)), full(vec(beta2b)),
            full(wm), full(vec(bm)), full(wv), full(vec(bv)),
            full(wx), full(vec(bx)),
        ],
        out_specs=(row_blk(d), row_blk(lz), row_blk(lz)),
        out_shape=(
            jax.ShapeDtypeStruct((n, d), F32),
            jax.ShapeDtypeStruct((n, lz), F32),
            jax.ShapeDtypeStruct((n, lz), F32),
        ),
    )(x, *sums_list, *cnts_list, onehot, u, eps,
      w1b, vec(b1b), w2a_x, w2a_agg, w2a_u, vec(b2a), vec(g2a), vec(beta2a),
      w2b, vec(b2b), vec(g2b), vec(beta2b), wm, vec(bm), wv, vec(bv),
      wx, vec(bx))
    return outs


def kernel(x, edge_index, edge_attr, u, batch, goal, W1a, b1a, g1, beta1,
           W1b, b1b, W2a, b2a, g2a, beta2a, W2b, b2b, g2b, beta2b,
           Wm, bm, Wv, bv, Wx, bx):
    n, d = x.shape
    row = edge_index[0]
    col = edge_index[1]

    w1a_x = W1a[:d]
    w1a_e = W1a[d:]
    w2a_x = W2a[:d]
    w2a_agg = W2a[d:d + W1b.shape[1]]
    w2a_u = W2a[d + W1b.shape[1]:]

    onehot = (batch[:, None] == jnp.arange(u.shape[0],
                                           dtype=batch.dtype)).astype(F32)
    eps = jax.random.normal(jax.random.key(42), (n, Wm.shape[1]), dtype=F32)

    xa = _stage_a(x, w1a_x)
    e = row.shape[0]
    nparts = 2
    part = e // nparts
    sums_list, cnts_list = [], []
    for k in range(nparts):
        lo = k * part
        g = _stage_b(xa, row, lo, part)
        r = _stage_c(g, edge_attr, w1a_e, b1a, g1, beta1, lo)
        s, c = _stage_d(r, col, n, lo)
        sums_list.append(s)
        cnts_list.append(c)
    out, z_mu, z_var = _stage_e(
        x, sums_list, cnts_list, onehot, u, eps, W1b, b1b, w2a_x, w2a_agg,
        w2a_u, b2a, g2a, beta2a, W2b, b2b, g2b, beta2b, Wm, bm, Wv, bv,
        Wx, bx)
    return (out, z_mu, z_var)
